# baseline (speedup 1.0000x reference)
import functools

import jax
import jax.numpy as jnp
from jax import lax
from jax.experimental import pallas as pl
from jax.experimental.pallas import tpu as pltpu
from jax.experimental.pallas import tpu_sc as plsc

N = 10000
NP = 10240
H = 16
NC = 2
NS = 16
NW = NC * NS
K = 80
CPT = 125
RPT = NP // NS
R = 8
D = 4
QD = 8

_sc_mesh = plsc.VectorSubcoreMesh(core_axis_name="c", subcore_axis_name="s")
_sc_params = pltpu.CompilerParams(use_tc_tiling_on_sc=False,
                                  skip_device_barrier=True)
_sc_params2 = pltpu.CompilerParams(use_tc_tiling_on_sc=False,
                                   needs_layout_passes=False,
                                   skip_device_barrier=True)
CPD = 2 * CPT


def _rsqrt_newton(d):
    i = plsc.bitcast(d, jnp.int32)
    i = jnp.int32(0x5F3759DF) - (i >> 1)
    y = plsc.bitcast(i, jnp.float32)
    half = d * (-0.5)
    for _ in range(3):
        y = y * (1.5 + half * y * y)
    return y


@functools.partial(
    pl.kernel,
    out_type=(
        jax.ShapeDtypeStruct((NC, NP, H), jnp.float32),
        jax.ShapeDtypeStruct((NP, H), jnp.float32),
        jax.ShapeDtypeStruct((NC, NP, H), jnp.float32),
    ),
    mesh=_sc_mesh,
    compiler_params=_sc_params2,
    scratch_types=[
        pltpu.VMEM((CPD, K), jnp.int32),
        pltpu.VMEM((CPT, K), jnp.int32),
        pltpu.VMEM((CPT, K), jnp.int32),
        pltpu.VMEM((R, K, H), jnp.float32),
        pltpu.VMEM((K, H), jnp.float32),
        pltpu.VMEM((RPT, H), jnp.float32),
        pltpu.VMEM((RPT, H), jnp.float32),
        pltpu.VMEM_SHARED((NP, H), jnp.float32),
        pltpu.VMEM_SHARED((NP, H), jnp.float32),
        pltpu.SemaphoreType.DMA,
        pltpu.SemaphoreType.DMA((R,)),
        pltpu.SemaphoreType.DMA((R,)),
    ],
)
def _layer1_kernel(eprop_hbm, h1_hbm, ones_hbm, zeros_hbm,
                   acc_hbm, dinv_hbm, gscr_hbm,
                   dstd_v, srcs_v, dsts_v, bufs, ones_v, dv_v, hg_v,
                   deg_sh, acc_sh, qsem, gsem, ssem):
    c = lax.axis_index("c")
    s = lax.axis_index("s")
    wid = c * NS + s
    r0 = s * RPT

    pltpu.sync_copy(zeros_hbm.at[pl.ds(r0, RPT)], dv_v)
    pltpu.sync_copy(dv_v, deg_sh.at[pl.ds(r0, RPT)])
    pltpu.sync_copy(dv_v, acc_sh.at[pl.ds(r0, RPT)])
    pltpu.sync_copy(ones_hbm, ones_v)
    pltpu.sync_copy(eprop_hbm.at[1, 2 * s], dstd_v.at[pl.ds(0, CPT)])
    pltpu.sync_copy(eprop_hbm.at[1, 2 * s + 1], dstd_v.at[pl.ds(CPT, CPT)])
    pltpu.sync_copy(eprop_hbm.at[0, wid], srcs_v)
    pltpu.sync_copy(eprop_hbm.at[1, wid], dsts_v)
    plsc.subcore_barrier()

    def d_issue(j):
        pltpu.async_copy(ones_v, deg_sh.at[dstd_v.at[j]], qsem, add=True)

    def d_drain():
        pltpu.make_async_copy(ones_v, deg_sh.at[dstd_v.at[0]], qsem).wait()

    for j in range(QD):
        d_issue(j)

    def d_body(i, carry):
        d_issue(i + QD)
        d_drain()
        return carry

    lax.fori_loop(0, CPD - QD, d_body, 0)
    for _ in range(QD):
        d_drain()
    plsc.subcore_barrier()

    pltpu.sync_copy(deg_sh.at[pl.ds(r0, RPT)], dv_v)
    pltpu.sync_copy(h1_hbm.at[pl.ds(r0, RPT)], hg_v)

    def nb_body(i, carry):
        for u in range(4):
            r = i * 4 + u
            y = _rsqrt_newton(dv_v[r] + 1.0)
            dv_v[r] = y
            hg_v[r] = hg_v[r] * y
        return carry

    lax.fori_loop(0, RPT // 4, nb_body, 0)
    pltpu.sync_copy(hg_v, gscr_hbm.at[c, pl.ds(r0, RPT)])

    @pl.when(c == 0)
    def _():
        pltpu.sync_copy(dv_v, dinv_hbm.at[pl.ds(r0, RPT)])

    plsc.subcore_barrier()

    def g_issue(j, b):
        pltpu.async_copy(gscr_hbm.at[c].at[srcs_v.at[j]], bufs.at[b],
                         gsem.at[b])

    def g_wait(b):
        pltpu.make_async_copy(gscr_hbm.at[c].at[srcs_v.at[0]], bufs.at[b],
                              gsem.at[b]).wait()

    def s_issue(j, b):
        pltpu.async_copy(bufs.at[b], acc_sh.at[dsts_v.at[j]], ssem.at[b],
                         add=True)

    def s_wait(b):
        pltpu.make_async_copy(bufs.at[b], acc_sh.at[dsts_v.at[0]],
                              ssem.at[b]).wait()

    _emit_prop_pipeline(g_issue, g_wait, s_issue, s_wait)
    plsc.subcore_barrier()
    pltpu.sync_copy(acc_sh.at[pl.ds(r0, RPT)], acc_hbm.at[c, pl.ds(r0, RPT)])


def _mm1_body(x_ref, w1_ref, h_ref):
    h = jnp.dot(x_ref[...], w1_ref[...], preferred_element_type=jnp.float32)
    h_ref[0:N] = h
    h_ref[N:NP] = jnp.zeros((NP - N, H), jnp.float32)


_mm1_call = pl.pallas_call(
    _mm1_body,
    out_shape=jax.ShapeDtypeStruct((NP, H), jnp.float32),
)


def _emit_prop_pipeline(g_issue, g_wait, s_issue, s_wait):
    for d in range(D):
        g_issue(d, d)
    for j in range(R):
        bn = (j + D) % R
        if j >= D:
            s_wait(bn)
        g_issue(j + D, bn)
        g_wait(j % R)
        s_issue(j, j % R)

    n_grp = (CPT - D) // R

    def group(i, carry):
        j0 = i * R
        for b in range(R):
            bn = (b + D) % R
            s_wait(bn)
            g_issue(j0 + b + D, bn)
            g_wait(b)
            s_issue(j0 + b, b)
        return carry

    lax.fori_loop(1, n_grp, group, 0)

    for j in range(n_grp * R, CPT):
        b = j % R
        bn = (b + D) % R
        s_wait(bn)
        if j + D < CPT:
            g_issue(j + D, bn)
        g_wait(b)
        s_issue(j, b)
    for j in range(CPT - D, CPT):
        s_wait(j % R)


@functools.partial(
    pl.kernel,
    out_type=jax.ShapeDtypeStruct((NC, NP, H), jnp.float32),
    mesh=_sc_mesh,
    compiler_params=_sc_params,
    scratch_types=[
        pltpu.VMEM((CPT, K), jnp.int32),
        pltpu.VMEM((K, H), jnp.float32),
        pltpu.VMEM((RPT, H), jnp.float32),
        pltpu.VMEM_SHARED((NP, H), jnp.float32),
        pltpu.SemaphoreType.DMA,
    ],
)
def _deg_kernel(eprop_hbm, ones_hbm, zeros_hbm, out_hbm,
                dsts_v, ones_v, z_v, acc_sh, ssem):
    c = lax.axis_index("c")
    s = lax.axis_index("s")
    wid = c * NS + s
    r0 = s * RPT
    pltpu.sync_copy(zeros_hbm.at[pl.ds(r0, RPT)], z_v)
    pltpu.sync_copy(z_v, acc_sh.at[pl.ds(r0, RPT)])
    pltpu.sync_copy(ones_hbm, ones_v)
    pltpu.sync_copy(eprop_hbm.at[1, wid], dsts_v)
    plsc.subcore_barrier()

    def s_issue(j):
        pltpu.async_copy(ones_v, acc_sh.at[dsts_v.at[j]], ssem, add=True)

    def s_drain():
        pltpu.make_async_copy(ones_v, acc_sh.at[dsts_v.at[0]], ssem).wait()

    for j in range(QD):
        s_issue(j)

    def body(i, carry):
        s_issue(i + QD)
        s_drain()
        return carry

    lax.fori_loop(0, CPT - QD, body, 0)
    for _ in range(QD):
        s_drain()
    plsc.subcore_barrier()
    pltpu.sync_copy(acc_sh.at[pl.ds(r0, RPT)], out_hbm.at[c, pl.ds(r0, RPT)])


@functools.partial(
    pl.kernel,
    out_type=jax.ShapeDtypeStruct((NC, NP, H), jnp.float32),
    mesh=_sc_mesh,
    compiler_params=_sc_params,
    scratch_types=[
        pltpu.VMEM((CPT, K), jnp.int32),
        pltpu.VMEM((CPT, K), jnp.int32),
        pltpu.VMEM((R, K, H), jnp.float32),
        pltpu.VMEM((RPT, H), jnp.float32),
        pltpu.VMEM_SHARED((NP, H), jnp.float32),
        pltpu.SemaphoreType.DMA((R,)),
        pltpu.SemaphoreType.DMA((R,)),
    ],
)
def _prop_kernel(eprop_hbm, g_hbm, zeros_hbm, out_hbm,
                 srcs_v, dsts_v, bufs, z_v, acc_sh, gsem, ssem):
    c = lax.axis_index("c")
    s = lax.axis_index("s")
    wid = c * NS + s
    r0 = s * RPT
    pltpu.sync_copy(zeros_hbm.at[pl.ds(r0, RPT)], z_v)
    pltpu.sync_copy(z_v, acc_sh.at[pl.ds(r0, RPT)])
    pltpu.sync_copy(eprop_hbm.at[0, wid], srcs_v)
    pltpu.sync_copy(eprop_hbm.at[1, wid], dsts_v)
    plsc.subcore_barrier()

    def g_issue(j, b):
        pltpu.async_copy(g_hbm.at[srcs_v.at[j]], bufs.at[b], gsem.at[b])

    def g_wait(b):
        pltpu.make_async_copy(g_hbm.at[srcs_v.at[0]], bufs.at[b],
                              gsem.at[b]).wait()

    def s_issue(j, b):
        pltpu.async_copy(bufs.at[b], acc_sh.at[dsts_v.at[j]], ssem.at[b],
                         add=True)

    def s_wait(b):
        pltpu.make_async_copy(bufs.at[b], acc_sh.at[dsts_v.at[0]],
                              ssem.at[b]).wait()

    _emit_prop_pipeline(g_issue, g_wait, s_issue, s_wait)
    plsc.subcore_barrier()
    pltpu.sync_copy(acc_sh.at[pl.ds(r0, RPT)], out_hbm.at[c, pl.ds(r0, RPT)])


def _pre_body(deg_ref, x_ref, w1_ref, g_ref, dinv_ref):
    deg = deg_ref[0] + deg_ref[1] + 1.0
    dinv = lax.rsqrt(deg)
    h = jnp.dot(x_ref[...], w1_ref[...], preferred_element_type=jnp.float32)
    g_ref[0:N] = h * dinv[0:N]
    g_ref[N:NP] = jnp.zeros((NP - N, H), jnp.float32)
    dinv_ref[...] = dinv


_pre_call = pl.pallas_call(
    _pre_body,
    out_shape=(
        jax.ShapeDtypeStruct((NP, H), jnp.float32),
        jax.ShapeDtypeStruct((NP, H), jnp.float32),
    ),
)


def _mid_body(acc_ref, h1_ref, dinv_ref, b1_ref, w2_ref, g2_ref):
    dinv = dinv_ref[...]
    p1 = dinv * (acc_ref[0] + acc_ref[1] + dinv * h1_ref[...]) + b1_ref[...]
    h1p = jnp.maximum(p1, 0.0)
    t2 = jnp.dot(h1p, w2_ref[...], preferred_element_type=jnp.float32)
    g2_ref[0:N] = t2[0:N] * dinv[0:N]
    g2_ref[N:NP] = jnp.zeros((NP - N, H), jnp.float32)


_mid_call = pl.pallas_call(
    _mid_body,
    out_shape=jax.ShapeDtypeStruct((NP, H), jnp.float32),
)


def _post_body(acc_ref, g2_ref, dinv_ref, b2_ref, out_ref):
    p2 = dinv_ref[0:N] * (acc_ref[0, 0:N] + acc_ref[1, 0:N] + g2_ref[0:N])
    p2 = p2 + b2_ref[...]
    m = jnp.max(p2, axis=1, keepdims=True)
    z = p2 - m
    lse = jnp.log(jnp.sum(jnp.exp(z), axis=1, keepdims=True))
    out_ref[...] = z - lse


_post_call = pl.pallas_call(
    _post_body,
    out_shape=jax.ShapeDtypeStruct((N, H), jnp.float32),
)


def kernel(x, edge_index, W1, b1, W2, b2):
    eprop = edge_index.astype(jnp.int32).reshape(2, NW, CPT, K)
    zeros2d = jnp.zeros((NP, H), jnp.float32)
    ones2d = jnp.ones((K, H), jnp.float32)

    h1 = _mm1_call(x, W1)
    acc1, dinv, _gs = _layer1_kernel(eprop, h1, ones2d, zeros2d)
    g2 = _mid_call(acc1, h1, dinv, jnp.broadcast_to(b1, (1, H)), W2)
    acc2 = _prop_kernel(eprop, g2, zeros2d)
    return _post_call(acc2, g2, dinv, jnp.broadcast_to(b2, (1, H)))


# speedup vs baseline: 1.0085x; 1.0085x over previous
import functools

import jax
import jax.numpy as jnp
from jax import lax
from jax.experimental import pallas as pl
from jax.experimental.pallas import tpu as pltpu
from jax.experimental.pallas import tpu_sc as plsc

N = 10000
NP = 10240
H = 16
NC = 2
NS = 16
NW = NC * NS
K = 80
CPT = 125
RPT = NP // NS
R = 8
D = 4
QD = 8

_sc_mesh = plsc.VectorSubcoreMesh(core_axis_name="c", subcore_axis_name="s")
_sc_params = pltpu.CompilerParams(use_tc_tiling_on_sc=False,
                                  skip_device_barrier=True)


def _emit_prop_pipeline(g_issue, g_wait, s_issue, s_wait):
    for d in range(D):
        g_issue(d, d)
    for j in range(R):
        bn = (j + D) % R
        if j >= D:
            s_wait(bn)
        g_issue(j + D, bn)
        g_wait(j % R)
        s_issue(j, j % R)

    n_grp = (CPT - D) // R

    def group(i, carry):
        j0 = i * R
        for b in range(R):
            bn = (b + D) % R
            s_wait(bn)
            g_issue(j0 + b + D, bn)
            g_wait(b)
            s_issue(j0 + b, b)
        return carry

    lax.fori_loop(1, n_grp, group, 0)

    for j in range(n_grp * R, CPT):
        b = j % R
        bn = (b + D) % R
        s_wait(bn)
        if j + D < CPT:
            g_issue(j + D, bn)
        g_wait(b)
        s_issue(j, b)
    for j in range(CPT - D, CPT):
        s_wait(j % R)


@functools.partial(
    pl.kernel,
    out_type=jax.ShapeDtypeStruct((NC, NP, H), jnp.float32),
    mesh=_sc_mesh,
    compiler_params=_sc_params,
    scratch_types=[
        pltpu.VMEM((CPT * K,), jnp.int32),
        pltpu.VMEM((K, H), jnp.float32),
        pltpu.VMEM((RPT, H), jnp.float32),
        pltpu.VMEM_SHARED((NP, H), jnp.float32),
        pltpu.SemaphoreType.DMA,
    ],
)
def _deg_kernel(eprop_hbm, ones_hbm, zeros_hbm, out_hbm,
                dsts_v, ones_v, z_v, acc_sh, ssem):
    c = lax.axis_index("c")
    s = lax.axis_index("s")
    wid = c * NS + s
    r0 = s * RPT
    pltpu.sync_copy(zeros_hbm.at[pl.ds(r0, RPT)], z_v)
    pltpu.sync_copy(z_v, acc_sh.at[pl.ds(r0, RPT)])
    pltpu.sync_copy(ones_hbm, ones_v)
    pltpu.sync_copy(eprop_hbm.at[1, pl.ds(wid * CPT * K, CPT * K)], dsts_v)
    plsc.subcore_barrier()

    def s_issue(j):
        pltpu.async_copy(ones_v, acc_sh.at[dsts_v.at[pl.ds(j * K, K)]], ssem,
                         add=True)

    def s_drain():
        pltpu.make_async_copy(ones_v, acc_sh.at[dsts_v.at[pl.ds(0, K)]],
                              ssem).wait()

    for j in range(QD):
        s_issue(j)

    def body(i, carry):
        s_issue(i + QD)
        s_drain()
        return carry

    lax.fori_loop(0, CPT - QD, body, 0)
    for _ in range(QD):
        s_drain()
    plsc.subcore_barrier()
    pltpu.sync_copy(acc_sh.at[pl.ds(r0, RPT)], out_hbm.at[c, pl.ds(r0, RPT)])


@functools.partial(
    pl.kernel,
    out_type=jax.ShapeDtypeStruct((NC, NP, H), jnp.float32),
    mesh=_sc_mesh,
    compiler_params=_sc_params,
    scratch_types=[
        pltpu.VMEM((CPT * K,), jnp.int32),
        pltpu.VMEM((CPT * K,), jnp.int32),
        pltpu.VMEM((R, K, H), jnp.float32),
        pltpu.VMEM((RPT, H), jnp.float32),
        pltpu.VMEM_SHARED((NP, H), jnp.float32),
        pltpu.SemaphoreType.DMA((R,)),
        pltpu.SemaphoreType.DMA((R,)),
    ],
)
def _prop_kernel(eprop_hbm, g_hbm, zeros_hbm, out_hbm,
                 srcs_v, dsts_v, bufs, z_v, acc_sh, gsem, ssem):
    c = lax.axis_index("c")
    s = lax.axis_index("s")
    wid = c * NS + s
    r0 = s * RPT
    pltpu.sync_copy(zeros_hbm.at[pl.ds(r0, RPT)], z_v)
    pltpu.sync_copy(z_v, acc_sh.at[pl.ds(r0, RPT)])
    pltpu.sync_copy(eprop_hbm.at[0, pl.ds(wid * CPT * K, CPT * K)], srcs_v)
    pltpu.sync_copy(eprop_hbm.at[1, pl.ds(wid * CPT * K, CPT * K)], dsts_v)
    plsc.subcore_barrier()

    def g_issue(j, b):
        pltpu.async_copy(g_hbm.at[srcs_v.at[pl.ds(j * K, K)]], bufs.at[b],
                         gsem.at[b])

    def g_wait(b):
        pltpu.make_async_copy(g_hbm.at[srcs_v.at[pl.ds(0, K)]], bufs.at[b],
                              gsem.at[b]).wait()

    def s_issue(j, b):
        pltpu.async_copy(bufs.at[b], acc_sh.at[dsts_v.at[pl.ds(j * K, K)]],
                         ssem.at[b], add=True)

    def s_wait(b):
        pltpu.make_async_copy(bufs.at[b], acc_sh.at[dsts_v.at[pl.ds(0, K)]],
                              ssem.at[b]).wait()

    _emit_prop_pipeline(g_issue, g_wait, s_issue, s_wait)
    plsc.subcore_barrier()
    pltpu.sync_copy(acc_sh.at[pl.ds(r0, RPT)], out_hbm.at[c, pl.ds(r0, RPT)])


def _pre_body(deg_ref, x_ref, w1_ref, g_ref, dinv_ref):
    deg = deg_ref[0] + deg_ref[1] + 1.0
    dinv = lax.rsqrt(deg)
    h = jnp.dot(x_ref[...], w1_ref[...], preferred_element_type=jnp.float32)
    g_ref[0:N] = h * dinv[0:N]
    g_ref[N:NP] = jnp.zeros((NP - N, H), jnp.float32)
    dinv_ref[...] = dinv


_pre_call = pl.pallas_call(
    _pre_body,
    out_shape=(
        jax.ShapeDtypeStruct((NP, H), jnp.float32),
        jax.ShapeDtypeStruct((NP, H), jnp.float32),
    ),
)


def _mid_body(acc_ref, g1_ref, dinv_ref, b1_ref, w2_ref, g2_ref):
    dinv = dinv_ref[...]
    p1 = dinv * (acc_ref[0] + acc_ref[1] + g1_ref[...]) + b1_ref[...]
    h1p = jnp.maximum(p1, 0.0)
    t2 = jnp.dot(h1p, w2_ref[...], preferred_element_type=jnp.float32)
    g2_ref[0:N] = t2[0:N] * dinv[0:N]
    g2_ref[N:NP] = jnp.zeros((NP - N, H), jnp.float32)


_mid_call = pl.pallas_call(
    _mid_body,
    out_shape=jax.ShapeDtypeStruct((NP, H), jnp.float32),
)


def _post_body(acc_ref, g2_ref, dinv_ref, b2_ref, out_ref):
    p2 = dinv_ref[0:N] * (acc_ref[0, 0:N] + acc_ref[1, 0:N] + g2_ref[0:N])
    p2 = p2 + b2_ref[...]
    m = jnp.max(p2, axis=1, keepdims=True)
    z = p2 - m
    lse = jnp.log(jnp.sum(jnp.exp(z), axis=1, keepdims=True))
    out_ref[...] = z - lse


_post_call = pl.pallas_call(
    _post_body,
    out_shape=jax.ShapeDtypeStruct((N, H), jnp.float32),
)


def kernel(x, edge_index, W1, b1, W2, b2):
    eprop = edge_index.astype(jnp.int32)
    zeros2d = jnp.zeros((NP, H), jnp.float32)
    ones2d = jnp.ones((K, H), jnp.float32)

    deg = _deg_kernel(eprop, ones2d, zeros2d)
    g1, dinv = _pre_call(deg, x, W1)
    acc1 = _prop_kernel(eprop, g1, zeros2d)
    g2 = _mid_call(acc1, g1, dinv, jnp.broadcast_to(b1, (1, H)), W2)
    acc2 = _prop_kernel(eprop, g2, zeros2d)
    return _post_call(acc2, g2, dinv, jnp.broadcast_to(b2, (1, H)))
